# Initial kernel scaffold; baseline (speedup 1.0000x reference)
import jax
import jax.numpy as jnp
from jax.experimental import pallas as pl


def kernel(x, w_in, b_in, w_out, b_out):
    raise NotImplementedError("write your pallas kernel here")


# speedup vs baseline: 2.4038x; 2.4038x over previous
import functools
import math

import jax
import jax.numpy as jnp
from jax import lax
from jax.experimental import pallas as pl
from jax.experimental.pallas import tpu as pltpu

_NEG_INF = -1e30


def _mhsa_kernel(x_ref, wqkv_ref, bqkv_ref, wo_ref, bo_ref, o_ref, *, n_heads):
    S = x_ref.shape[1]
    D = x_ref.shape[2]
    dh = D // n_heads

    x = x_ref[0].astype(jnp.bfloat16)
    qkv = lax.dot_general(
        x, wqkv_ref[...], (((1,), (0,)), ((), ())),
        preferred_element_type=jnp.float32) + bqkv_ref[...]

    qi = lax.broadcasted_iota(jnp.int32, (S, S), 0)
    ki = lax.broadcasted_iota(jnp.int32, (S, S), 1)
    causal = ki <= qi

    heads = []
    for h in range(n_heads):
        q = qkv[:, h * dh:(h + 1) * dh].astype(jnp.bfloat16)
        k = qkv[:, D + h * dh:D + (h + 1) * dh].astype(jnp.bfloat16)
        v = qkv[:, 2 * D + h * dh:2 * D + (h + 1) * dh].astype(jnp.bfloat16)
        s = lax.dot_general(q, k, (((1,), (1,)), ((), ())),
                            preferred_element_type=jnp.float32)
        s = jnp.where(causal, s, _NEG_INF)
        m = jnp.max(s, axis=-1, keepdims=True)
        p = jnp.exp(s - m)
        l = jnp.sum(p, axis=-1, keepdims=True)
        o = lax.dot_general(p.astype(jnp.bfloat16), v,
                            (((1,), (0,)), ((), ())),
                            preferred_element_type=jnp.float32)
        heads.append((o / l).astype(jnp.bfloat16))

    attn = jnp.concatenate(heads, axis=1)
    out = lax.dot_general(attn, wo_ref[...], (((1,), (0,)), ((), ())),
                          preferred_element_type=jnp.float32) + bo_ref[...]
    o_ref[0] = out.astype(o_ref.dtype)


def kernel(x, w_in, b_in, w_out, b_out):
    B, S, D = x.shape
    H = 12
    dh = D // H
    scale = 1.0 / math.sqrt(dh)

    scale_vec = jnp.concatenate([
        jnp.full((D,), scale, jnp.float32),
        jnp.ones((2 * D,), jnp.float32)])
    w_qkv = (jnp.transpose(w_in) * scale_vec[None, :]).astype(jnp.bfloat16)
    b_qkv = (b_in * scale_vec).reshape(1, 3 * D).astype(jnp.float32)
    wo = jnp.transpose(w_out).astype(jnp.bfloat16)
    bo = b_out.reshape(1, D).astype(jnp.float32)

    return pl.pallas_call(
        functools.partial(_mhsa_kernel, n_heads=H),
        out_shape=jax.ShapeDtypeStruct((B, S, D), x.dtype),
        grid=(B,),
        in_specs=[
            pl.BlockSpec((1, S, D), lambda b: (b, 0, 0)),
            pl.BlockSpec((D, 3 * D), lambda b: (0, 0)),
            pl.BlockSpec((1, 3 * D), lambda b: (0, 0)),
            pl.BlockSpec((D, D), lambda b: (0, 0)),
            pl.BlockSpec((1, D), lambda b: (0, 0)),
        ],
        out_specs=pl.BlockSpec((1, S, D), lambda b: (b, 0, 0)),
        compiler_params=pltpu.CompilerParams(
            dimension_semantics=("parallel",),
            vmem_limit_bytes=(56 << 20)),
    )(x, w_qkv, b_qkv, wo, bo)


# speedup vs baseline: 2.7565x; 1.1467x over previous
import functools
import math

import jax
import jax.numpy as jnp
from jax import lax
from jax.experimental import pallas as pl
from jax.experimental.pallas import tpu as pltpu

_NEG_INF = -1e30


def _mhsa_kernel(x_ref, wqkv_ref, bqkv_ref, wo_ref, bo_ref, o_ref, *,
                 n_heads, scale):
    S = x_ref.shape[1]
    D = x_ref.shape[2]
    dh = D // n_heads

    x = x_ref[0].astype(jnp.bfloat16)
    qkv = lax.dot_general(
        x, wqkv_ref[...], (((1,), (1,)), ((), ())),
        preferred_element_type=jnp.float32) + bqkv_ref[...]

    qi = lax.broadcasted_iota(jnp.int32, (S, S), 0)
    ki = lax.broadcasted_iota(jnp.int32, (S, S), 1)
    causal = ki <= qi

    heads = []
    for h in range(n_heads):
        q = (qkv[:, h * dh:(h + 1) * dh] * scale).astype(jnp.bfloat16)
        k = qkv[:, D + h * dh:D + (h + 1) * dh].astype(jnp.bfloat16)
        v = qkv[:, 2 * D + h * dh:2 * D + (h + 1) * dh].astype(jnp.bfloat16)
        s = lax.dot_general(q, k, (((1,), (1,)), ((), ())),
                            preferred_element_type=jnp.float32)
        s = jnp.where(causal, s, _NEG_INF)
        m = jnp.max(s, axis=-1, keepdims=True)
        p = jnp.exp(s - m)
        l = jnp.sum(p, axis=-1, keepdims=True)
        o = lax.dot_general(p.astype(jnp.bfloat16), v,
                            (((1,), (0,)), ((), ())),
                            preferred_element_type=jnp.float32)
        heads.append((o / l).astype(jnp.bfloat16))

    attn = jnp.concatenate(heads, axis=1)
    out = lax.dot_general(attn, wo_ref[...], (((1,), (1,)), ((), ())),
                          preferred_element_type=jnp.float32) + bo_ref[...]
    o_ref[0] = out.astype(o_ref.dtype)


def kernel(x, w_in, b_in, w_out, b_out):
    B, S, D = x.shape
    H = 12
    dh = D // H
    scale = 1.0 / math.sqrt(dh)

    w_qkv = w_in.astype(jnp.bfloat16)
    b_qkv = b_in.reshape(1, 3 * D)
    wo = w_out.astype(jnp.bfloat16)
    bo = b_out.reshape(1, D)

    return pl.pallas_call(
        functools.partial(_mhsa_kernel, n_heads=H, scale=scale),
        out_shape=jax.ShapeDtypeStruct((B, S, D), x.dtype),
        grid=(B,),
        in_specs=[
            pl.BlockSpec((1, S, D), lambda b: (b, 0, 0)),
            pl.BlockSpec((3 * D, D), lambda b: (0, 0)),
            pl.BlockSpec((1, 3 * D), lambda b: (0, 0)),
            pl.BlockSpec((D, D), lambda b: (0, 0)),
            pl.BlockSpec((1, D), lambda b: (0, 0)),
        ],
        out_specs=pl.BlockSpec((1, S, D), lambda b: (b, 0, 0)),
        compiler_params=pltpu.CompilerParams(
            dimension_semantics=("parallel",),
            vmem_limit_bytes=(56 << 20)),
    )(x, w_qkv, b_qkv, wo, bo)
